# Initial kernel scaffold; baseline (speedup 1.0000x reference)
import jax
import jax.numpy as jnp
from jax.experimental import pallas as pl


def kernel(enc_prob, enc_len, ngram_table):
    raise NotImplementedError("write your pallas kernel here")


# speedup vs baseline: 1.2031x; 1.2031x over previous
import functools

import jax
import jax.numpy as jnp
from jax.experimental import pallas as pl
from jax.experimental.pallas import tpu as pltpu

EPS = 1e-10
NEG_LOG_EPS = 23.025850929940457


def _kld_block(x_ref, len_ref, w_ref, tab_ref, out_ref, neglog_ref, carry_ref,
               *, tb: int, v: int, nt: int):
    b = pl.program_id(0)
    t = pl.program_id(1)

    @pl.when(jnp.logical_and(b == 0, t == 0))
    def _init_table():
        neglog_ref[...] = -jnp.log(tab_ref[...])

    @pl.when(jnp.logical_and(b == 0, t == 0))
    def _init_out():
        out_ref[...] = jnp.zeros((1, 1), jnp.float32)

    x = x_ref[0]

    maxv = jnp.max(x, axis=-1, keepdims=True)
    lane = jax.lax.broadcasted_iota(jnp.int32, (tb, v), 1)
    amax = jnp.min(jnp.where(x == maxv, lane, v),
                   axis=-1, keepdims=True)

    prev = jnp.where(t == 0, 1, carry_ref[0])
    rolled = jnp.roll(amax, 1, axis=0)
    row = jax.lax.broadcasted_iota(jnp.int32, (tb, 1), 0)
    codes = jnp.where(row == 0, prev, rolled)
    carry_ref[0] = amax[tb - 1, 0]

    onehot = (codes == jax.lax.broadcasted_iota(jnp.int32, (tb, v), 1))
    g = jax.lax.dot_general(onehot.astype(jnp.float32), neglog_ref[...],
                            (((1,), (0,)), ((), ())),
                            preferred_element_type=jnp.float32)

    dots = jnp.sum(x * g, axis=-1, keepdims=True)
    rowsum = jnp.sum(x, axis=-1, keepdims=True)

    tpos = t * tb + row
    masked = tpos >= len_ref[b]
    val = jnp.where(masked, NEG_LOG_EPS * rowsum, dots)
    out_ref[...] += jnp.full((1, 1), jnp.sum(val) * w_ref[b], jnp.float32)


def kernel(enc_prob, enc_len, ngram_table):
    B, T, V = enc_prob.shape
    TB = 256
    NT = T // TB

    lens = enc_len.astype(jnp.int32)
    w = 1.0 / (enc_len.astype(jnp.float32) * B)

    out = pl.pallas_call(
        functools.partial(_kld_block, tb=TB, v=V, nt=NT),
        grid=(B, NT),
        in_specs=[
            pl.BlockSpec((1, TB, V), lambda b, t: (b, t, 0)),
            pl.BlockSpec(memory_space=pltpu.SMEM),
            pl.BlockSpec(memory_space=pltpu.SMEM),
            pl.BlockSpec((V, V), lambda b, t: (0, 0)),
        ],
        out_specs=pl.BlockSpec((1, 1), lambda b, t: (0, 0)),
        out_shape=jax.ShapeDtypeStruct((1, 1), jnp.float32),
        scratch_shapes=[
            pltpu.VMEM((V, V), jnp.float32),
            pltpu.SMEM((1,), jnp.int32),
        ],
    )(enc_prob, lens, w, ngram_table)
    return out[0, 0]
